# Initial kernel scaffold; baseline (speedup 1.0000x reference)
import jax
import jax.numpy as jnp
from jax.experimental import pallas as pl


def kernel(x, edge_index, Wl1, bl1, Wr1, Wl2, bl2, Wr2):
    raise NotImplementedError("write your pallas kernel here")


# speedup vs baseline: 5.4802x; 5.4802x over previous
import functools

import jax
import jax.numpy as jnp
from jax import lax
from jax.experimental import pallas as pl
from jax.experimental.pallas import tpu as pltpu
from jax.experimental.pallas import tpu_sc as plsc

_N = 10000
_E = 320000
_D_IN = 128
_D_HID = 128
_D_OUT = 64

_NC = 2
_NS = 16
_NW = _NC * _NS
_EPW = _E // _NW
_CH = 80
_NCHUNK = _EPW // _CH
_RPT = 624
_TAIL = _N - _NS * _RPT
_TOFF = _NS * _RPT

_BLK = 2000
_GRID = _N // _BLK


def _make_sc_agg(D, with_count):
  out_type = [jax.ShapeDtypeStruct((_NC, _N, D), jnp.float32)]
  scratch = [
      pltpu.VMEM((_CH,), jnp.int32),
      pltpu.VMEM((_CH,), jnp.int32),
      pltpu.VMEM((_CH, D), jnp.float32),
      pltpu.VMEM_SHARED((_N, D), jnp.float32),
      pltpu.SemaphoreType.DMA,
  ]
  if with_count:
    out_type.append(jax.ShapeDtypeStruct((_NW, 1, _N), jnp.float32))
    scratch.append(pltpu.VMEM((1, _N), jnp.float32))

  mesh = plsc.VectorSubcoreMesh(
      core_axis_name="c", subcore_axis_name="s",
      num_cores=_NC, num_subcores=_NS)

  def body(*refs):
    if with_count:
      (p_hbm, src_hbm, dst_hbm, z_hbm, zc_hbm,
       out_hbm, cnt_hbm, sidx, didx, rows, acc, sem, cnt) = refs
    else:
      (p_hbm, src_hbm, dst_hbm, z_hbm,
       out_hbm, sidx, didx, rows, acc, sem) = refs

    cid = lax.axis_index("c")
    sid = lax.axis_index("s")
    wid = cid * _NS + sid
    row0 = sid * _RPT

    pltpu.sync_copy(z_hbm, acc.at[pl.ds(row0, _RPT)])
    if with_count:
      pltpu.sync_copy(zc_hbm, cnt)

    @pl.when(sid == 0)
    def _zero_tail():
      pltpu.sync_copy(z_hbm.at[pl.ds(0, _TAIL)], acc.at[pl.ds(_TOFF, _TAIL)])

    plsc.subcore_barrier()

    def step(i, carry):
      base = pl.multiple_of(wid * _EPW + i * _CH, 8)
      pltpu.sync_copy(src_hbm.at[pl.ds(base, _CH)], sidx)
      pltpu.sync_copy(dst_hbm.at[pl.ds(base, _CH)], didx)
      pltpu.async_copy(p_hbm.at[sidx], rows, sem).wait()
      pltpu.sync_copy(rows, acc.at[didx], add=True)
      if with_count:
        zero16 = jnp.zeros((16,), jnp.int32)
        one16 = jnp.ones((16,), jnp.float32)
        for g in range(_CH // 16):
          iv = didx[pl.ds(g * 16, 16)]
          plsc.addupdate_scatter(cnt, [zero16, iv], one16)
      return carry

    lax.fori_loop(0, _NCHUNK, step, 0)
    plsc.subcore_barrier()

    pltpu.sync_copy(acc.at[pl.ds(row0, _RPT)],
                    out_hbm.at[cid, pl.ds(row0, _RPT)])
    if with_count:
      pltpu.sync_copy(cnt, cnt_hbm.at[wid])

    @pl.when(sid == 0)
    def _write_tail():
      pltpu.sync_copy(acc.at[pl.ds(_TOFF, _TAIL)],
                      out_hbm.at[cid, pl.ds(_TOFF, _TAIL)])

  kw = {}
  if with_count:
    kw["compiler_params"] = pltpu.CompilerParams(needs_layout_passes=False)
  return functools.partial(
      pl.kernel,
      out_type=out_type if with_count else out_type[0],
      mesh=mesh,
      scratch_types=scratch,
      **kw,
  )(body)


@functools.lru_cache(maxsize=None)
def _sc_agg(D, with_count):
  return _make_sc_agg(D, with_count)


def _tc1_body(x_ref, wl_ref, wr_ref, bl_ref, p_ref, r_ref):
  xv = x_ref[...]
  p_ref[...] = lax.dot_general(xv, wl_ref[...], (((1,), (1,)), ((), ())),
                               preferred_element_type=jnp.float32)
  r_ref[...] = lax.dot_general(xv, wr_ref[...], (((1,), (1,)), ((), ())),
                               preferred_element_type=jnp.float32) + bl_ref[...]


_tc1 = pl.pallas_call(
    _tc1_body,
    grid=(_GRID,),
    in_specs=[
        pl.BlockSpec((_BLK, _D_IN), lambda i: (i, 0)),
        pl.BlockSpec((_D_HID, _D_IN), lambda i: (0, 0)),
        pl.BlockSpec((_D_HID, _D_IN), lambda i: (0, 0)),
        pl.BlockSpec((1, _D_HID), lambda i: (0, 0)),
    ],
    out_specs=[
        pl.BlockSpec((_BLK, _D_HID), lambda i: (i, 0)),
        pl.BlockSpec((_BLK, _D_HID), lambda i: (i, 0)),
    ],
    out_shape=[
        jax.ShapeDtypeStruct((_N, _D_HID), jnp.float32),
        jax.ShapeDtypeStruct((_N, _D_HID), jnp.float32),
    ],
)


def _tc_cnt_body(cnt_ref, inv_ref):
  total = jnp.sum(cnt_ref[...], axis=0, keepdims=True)
  inv = 1.0 / jnp.maximum(total, 1.0)
  ones = jnp.ones((1, _D_HID), jnp.float32)
  inv_ref[...] = lax.dot_general(inv, ones, (((0,), (0,)), ((), ())),
                                 preferred_element_type=jnp.float32)


_tc_cnt = pl.pallas_call(
    _tc_cnt_body,
    in_specs=[pl.BlockSpec((_NW, _N), lambda: (0, 0))],
    out_specs=pl.BlockSpec((_N, _D_HID), lambda: (0, 0)),
    out_shape=jax.ShapeDtypeStruct((_N, _D_HID), jnp.float32),
)


def _tc2_body(agg_ref, inv_ref, r1_ref, h_ref):
  mean = (agg_ref[0] + agg_ref[1]) * inv_ref[...]
  h_ref[...] = jnp.maximum(mean + r1_ref[...], 0.0)


_tc2 = pl.pallas_call(
    _tc2_body,
    grid=(_GRID,),
    in_specs=[
        pl.BlockSpec((_NC, _BLK, _D_HID), lambda i: (0, i, 0)),
        pl.BlockSpec((_BLK, _D_HID), lambda i: (i, 0)),
        pl.BlockSpec((_BLK, _D_HID), lambda i: (i, 0)),
    ],
    out_specs=pl.BlockSpec((_BLK, _D_HID), lambda i: (i, 0)),
    out_shape=jax.ShapeDtypeStruct((_N, _D_HID), jnp.float32),
)


def _tc3_body(agg_ref, inv_ref, h_ref, wl_ref, wr_ref, bl_ref, out_ref):
  mean = (agg_ref[0] + agg_ref[1]) * inv_ref[...]
  out_ref[...] = (
      lax.dot_general(mean, wl_ref[...], (((1,), (1,)), ((), ())),
                      preferred_element_type=jnp.float32)
      + lax.dot_general(h_ref[...], wr_ref[...], (((1,), (1,)), ((), ())),
                        preferred_element_type=jnp.float32)
      + bl_ref[...])


_tc3 = pl.pallas_call(
    _tc3_body,
    grid=(_GRID,),
    in_specs=[
        pl.BlockSpec((_NC, _BLK, _D_HID), lambda i: (0, i, 0)),
        pl.BlockSpec((_BLK, _D_HID), lambda i: (i, 0)),
        pl.BlockSpec((_BLK, _D_HID), lambda i: (i, 0)),
        pl.BlockSpec((_D_OUT, _D_HID), lambda i: (0, 0)),
        pl.BlockSpec((_D_OUT, _D_HID), lambda i: (0, 0)),
        pl.BlockSpec((1, _D_OUT), lambda i: (0, 0)),
    ],
    out_specs=pl.BlockSpec((_BLK, _D_OUT), lambda i: (i, 0)),
    out_shape=jax.ShapeDtypeStruct((_N, _D_OUT), jnp.float32),
)


def kernel(x, edge_index, Wl1, bl1, Wr1, Wl2, bl2, Wr2):
  src = edge_index[0]
  dst = edge_index[1]

  z128 = jnp.zeros((_RPT, _D_HID), jnp.float32)
  zc = jnp.zeros((1, _N), jnp.float32)

  p1, r1 = _tc1(x, Wl1, Wr1, bl1.reshape(1, -1))
  agg1, cnt = _sc_agg(_D_HID, True)(p1, src, dst, z128, zc)
  invb = _tc_cnt(cnt.reshape(_NW, _N))
  h = _tc2(agg1, invb, r1)
  agg2 = _sc_agg(_D_HID, False)(h, src, dst, z128)
  return _tc3(agg2, invb, h, Wl2, Wr2, bl2.reshape(1, -1))


# speedup vs baseline: 10.3278x; 1.8846x over previous
import functools

import jax
import jax.numpy as jnp
from jax import lax
from jax.experimental import pallas as pl
from jax.experimental.pallas import tpu as pltpu
from jax.experimental.pallas import tpu_sc as plsc

_N = 10000
_E = 320000
_D_IN = 128
_D_HID = 128
_D_OUT = 64

_NC = 2
_NS = 16
_NW = _NC * _NS
_EPW = _E // _NW
_CH = 80
_NCHUNK = _EPW // _CH
_NJ = _NCHUNK // 2
_RPT = 624
_TAIL = _N - _NS * _RPT
_TOFF = _NS * _RPT

_BLK = 2000
_GRID = _N // _BLK


def _make_sc_agg(D, with_count):
  out_type = [jax.ShapeDtypeStruct((_NC, _N, D), jnp.float32)]
  scratch = [
      pltpu.VMEM((_CH,), jnp.int32),
      pltpu.VMEM((_CH,), jnp.int32),
      pltpu.VMEM((_CH,), jnp.int32),
      pltpu.VMEM((_CH,), jnp.int32),
      pltpu.VMEM((_CH, D), jnp.float32),
      pltpu.VMEM((_CH, D), jnp.float32),
      pltpu.VMEM_SHARED((_N, D), jnp.float32),
      pltpu.SemaphoreType.DMA,
      pltpu.SemaphoreType.DMA,
      pltpu.SemaphoreType.DMA,
      pltpu.SemaphoreType.DMA,
      pltpu.SemaphoreType.DMA,
      pltpu.SemaphoreType.DMA,
  ]
  if with_count:
    out_type.append(jax.ShapeDtypeStruct((_NW, 1, _N), jnp.float32))
    scratch.append(pltpu.VMEM((1, _N), jnp.float32))

  mesh = plsc.VectorSubcoreMesh(
      core_axis_name="c", subcore_axis_name="s",
      num_cores=_NC, num_subcores=_NS)

  def body(*refs):
    if with_count:
      (p_hbm, src_hbm, dst_hbm, z_hbm, zc_hbm,
       out_hbm, cnt_hbm,
       si0, di0, si1, di1, rows0, rows1, acc,
       is0, is1, gs0, gs1, ss0, ss1, cnt) = refs
    else:
      (p_hbm, src_hbm, dst_hbm, z_hbm,
       out_hbm,
       si0, di0, si1, di1, rows0, rows1, acc,
       is0, is1, gs0, gs1, ss0, ss1) = refs

    cid = lax.axis_index("c")
    sid = lax.axis_index("s")
    wid = cid * _NS + sid
    row0 = sid * _RPT

    if with_count:
      pltpu.sync_copy(zc_hbm, cnt)
    pltpu.sync_copy(z_hbm, acc.at[pl.ds(row0, _RPT)])

    @pl.when(sid == 0)
    def _zero_tail():
      pltpu.sync_copy(z_hbm.at[pl.ds(0, _TAIL)], acc.at[pl.ds(_TOFF, _TAIL)])

    plsc.subcore_barrier()

    def _idx(i, si, di, sem):
      base = pl.multiple_of(wid * _EPW + i * _CH, 8)
      pltpu.async_copy(src_hbm.at[pl.ds(base, _CH)], si, sem)
      pltpu.async_copy(dst_hbm.at[pl.ds(base, _CH)], di, sem)

    def _iwait(si, di, sem):
      pltpu.make_async_copy(src_hbm.at[pl.ds(0, _CH)], si, sem).wait()
      pltpu.make_async_copy(src_hbm.at[pl.ds(0, _CH)], di, sem).wait()

    def _gather(si, buf, sem):
      pltpu.async_copy(p_hbm.at[si], buf, sem)

    def _gwait(si, buf, sem):
      pltpu.make_async_copy(p_hbm.at[si], buf, sem).wait()

    def _scatter(di, buf, sem):
      pltpu.make_async_copy(buf, acc.at[di], sem).start(add=True)

    def _swait(di, buf, sem):
      pltpu.make_async_copy(buf, acc.at[di], sem).wait()

    def _counts(di):
      if with_count:
        zero16 = jnp.zeros((16,), jnp.int32)
        one16 = jnp.ones((16,), jnp.float32)
        for g in range(_CH // 16):
          iv = di[pl.ds(g * 16, 16)]
          plsc.addupdate_scatter(cnt, [zero16, iv], one16)

    _idx(0, si0, di0, is0)
    _iwait(si0, di0, is0)
    _gather(si0, rows0, gs0)

    def step(j, carry):
      a = j * 2

      @pl.when(j >= 1)
      def _():
        _swait(di1, rows1, ss1)
      _idx(a + 1, si1, di1, is1)
      _counts(di0)
      _iwait(si1, di1, is1)
      _gather(si1, rows1, gs1)
      _gwait(si0, rows0, gs0)
      _scatter(di0, rows0, ss0)

      @pl.when(j < _NJ - 1)
      def _():
        _swait(di0, rows0, ss0)
        _idx(a + 2, si0, di0, is0)
      _counts(di1)

      @pl.when(j < _NJ - 1)
      def _():
        _iwait(si0, di0, is0)
        _gather(si0, rows0, gs0)
      _gwait(si1, rows1, gs1)
      _scatter(di1, rows1, ss1)
      return carry

    lax.fori_loop(0, _NJ, step, 0)

    _swait(di0, rows0, ss0)
    _idx(_NCHUNK - 1, si0, di0, is0)
    _iwait(si0, di0, is0)
    _gather(si0, rows0, gs0)
    _counts(di0)
    _gwait(si0, rows0, gs0)
    _scatter(di0, rows0, ss0)
    _swait(di0, rows0, ss0)
    _swait(di1, rows1, ss1)
    plsc.subcore_barrier()

    pltpu.sync_copy(acc.at[pl.ds(row0, _RPT)],
                    out_hbm.at[cid, pl.ds(row0, _RPT)])
    if with_count:
      pltpu.sync_copy(cnt, cnt_hbm.at[wid])

    @pl.when(sid == 0)
    def _write_tail():
      pltpu.sync_copy(acc.at[pl.ds(_TOFF, _TAIL)],
                      out_hbm.at[cid, pl.ds(_TOFF, _TAIL)])

  kw = {}
  if with_count:
    kw["compiler_params"] = pltpu.CompilerParams(needs_layout_passes=False)
  return functools.partial(
      pl.kernel,
      out_type=out_type if with_count else out_type[0],
      mesh=mesh,
      scratch_types=scratch,
      **kw,
  )(body)


@functools.lru_cache(maxsize=None)
def _sc_agg(D, with_count):
  return _make_sc_agg(D, with_count)


def _tc1_body(x_ref, wl_ref, wr_ref, bl_ref, p_ref, r_ref):
  xv = x_ref[...]
  p_ref[...] = lax.dot_general(xv, wl_ref[...], (((1,), (1,)), ((), ())),
                               preferred_element_type=jnp.float32)
  r_ref[...] = lax.dot_general(xv, wr_ref[...], (((1,), (1,)), ((), ())),
                               preferred_element_type=jnp.float32) + bl_ref[...]


_tc1 = pl.pallas_call(
    _tc1_body,
    grid=(_GRID,),
    in_specs=[
        pl.BlockSpec((_BLK, _D_IN), lambda i: (i, 0)),
        pl.BlockSpec((_D_HID, _D_IN), lambda i: (0, 0)),
        pl.BlockSpec((_D_HID, _D_IN), lambda i: (0, 0)),
        pl.BlockSpec((1, _D_HID), lambda i: (0, 0)),
    ],
    out_specs=[
        pl.BlockSpec((_BLK, _D_HID), lambda i: (i, 0)),
        pl.BlockSpec((_BLK, _D_HID), lambda i: (i, 0)),
    ],
    out_shape=[
        jax.ShapeDtypeStruct((_N, _D_HID), jnp.float32),
        jax.ShapeDtypeStruct((_N, _D_HID), jnp.float32),
    ],
)


def _tc_cnt_body(cnt_ref, inv_ref):
  total = jnp.sum(cnt_ref[...], axis=0, keepdims=True)
  inv = 1.0 / jnp.maximum(total, 1.0)
  ones = jnp.ones((1, _D_HID), jnp.float32)
  inv_ref[...] = lax.dot_general(inv, ones, (((0,), (0,)), ((), ())),
                                 preferred_element_type=jnp.float32)


_tc_cnt = pl.pallas_call(
    _tc_cnt_body,
    in_specs=[pl.BlockSpec((_NW, _N), lambda: (0, 0))],
    out_specs=pl.BlockSpec((_N, _D_HID), lambda: (0, 0)),
    out_shape=jax.ShapeDtypeStruct((_N, _D_HID), jnp.float32),
)


def _tc2_body(agg_ref, inv_ref, r1_ref, h_ref):
  mean = (agg_ref[0] + agg_ref[1]) * inv_ref[...]
  h_ref[...] = jnp.maximum(mean + r1_ref[...], 0.0)


_tc2 = pl.pallas_call(
    _tc2_body,
    grid=(_GRID,),
    in_specs=[
        pl.BlockSpec((_NC, _BLK, _D_HID), lambda i: (0, i, 0)),
        pl.BlockSpec((_BLK, _D_HID), lambda i: (i, 0)),
        pl.BlockSpec((_BLK, _D_HID), lambda i: (i, 0)),
    ],
    out_specs=pl.BlockSpec((_BLK, _D_HID), lambda i: (i, 0)),
    out_shape=jax.ShapeDtypeStruct((_N, _D_HID), jnp.float32),
)


def _tc3_body(agg_ref, inv_ref, h_ref, wl_ref, wr_ref, bl_ref, out_ref):
  mean = (agg_ref[0] + agg_ref[1]) * inv_ref[...]
  out_ref[...] = (
      lax.dot_general(mean, wl_ref[...], (((1,), (1,)), ((), ())),
                      preferred_element_type=jnp.float32)
      + lax.dot_general(h_ref[...], wr_ref[...], (((1,), (1,)), ((), ())),
                        preferred_element_type=jnp.float32)
      + bl_ref[...])


_tc3 = pl.pallas_call(
    _tc3_body,
    grid=(_GRID,),
    in_specs=[
        pl.BlockSpec((_NC, _BLK, _D_HID), lambda i: (0, i, 0)),
        pl.BlockSpec((_BLK, _D_HID), lambda i: (i, 0)),
        pl.BlockSpec((_BLK, _D_HID), lambda i: (i, 0)),
        pl.BlockSpec((_D_OUT, _D_HID), lambda i: (0, 0)),
        pl.BlockSpec((_D_OUT, _D_HID), lambda i: (0, 0)),
        pl.BlockSpec((1, _D_OUT), lambda i: (0, 0)),
    ],
    out_specs=pl.BlockSpec((_BLK, _D_OUT), lambda i: (i, 0)),
    out_shape=jax.ShapeDtypeStruct((_N, _D_OUT), jnp.float32),
)


def kernel(x, edge_index, Wl1, bl1, Wr1, Wl2, bl2, Wr2):
  src = edge_index[0]
  dst = edge_index[1]

  z128 = jnp.zeros((_RPT, _D_HID), jnp.float32)
  zc = jnp.zeros((1, _N), jnp.float32)

  p1, r1 = _tc1(x, Wl1, Wr1, bl1.reshape(1, -1))
  agg1, cnt = _sc_agg(_D_HID, True)(p1, src, dst, z128, zc)
  invb = _tc_cnt(cnt.reshape(_NW, _N))
  h = _tc2(agg1, invb, r1)
  agg2 = _sc_agg(_D_HID, False)(h, src, dst, z128)
  return _tc3(agg2, invb, h, Wl2, Wr2, bl2.reshape(1, -1))


# speedup vs baseline: 12.1087x; 1.1724x over previous
import functools

import jax
import jax.numpy as jnp
from jax import lax
from jax.experimental import pallas as pl
from jax.experimental.pallas import tpu as pltpu
from jax.experimental.pallas import tpu_sc as plsc

_N = 10000
_E = 320000
_D_IN = 128
_D_HID = 128
_D_OUT = 64

_NC = 2
_NS = 16
_NW = _NC * _NS
_EPW = _E // _NW
_CH = 80
_NCHUNK = _EPW // _CH
_NJ = _NCHUNK // 2
_RPT = 624
_TAIL = _N - _NS * _RPT
_TOFF = _NS * _RPT

_BLK = 2000
_GRID = _N // _BLK


def _make_sc_agg(D, with_count):
  out_type = [jax.ShapeDtypeStruct((_NC, _N, D), jnp.float32)]
  scratch = (
      [pltpu.VMEM((_CH,), jnp.int32)] * 8
      + [pltpu.VMEM((_CH, D), jnp.float32)] * 2
      + [pltpu.VMEM_SHARED((_N, D), jnp.float32)]
      + [pltpu.SemaphoreType.DMA] * 8
  )
  if with_count:
    out_type.append(jax.ShapeDtypeStruct((_NW, 1, _N), jnp.float32))
    scratch.append(pltpu.VMEM((1, _N), jnp.float32))

  mesh = plsc.VectorSubcoreMesh(
      core_axis_name="c", subcore_axis_name="s",
      num_cores=_NC, num_subcores=_NS)

  def body(*refs):
    if with_count:
      (p_hbm, src_hbm, dst_hbm, z_hbm, zc_hbm,
       out_hbm, cnt_hbm,
       si0, di0, si1, di1, si2, di2, si3, di3, rows0, rows1, acc,
       is0, is1, is2, is3, gs0, gs1, ss0, ss1, cnt) = refs
    else:
      (p_hbm, src_hbm, dst_hbm, z_hbm,
       out_hbm,
       si0, di0, si1, di1, si2, di2, si3, di3, rows0, rows1, acc,
       is0, is1, is2, is3, gs0, gs1, ss0, ss1) = refs
    sets = [(si0, di0, is0), (si1, di1, is1),
            (si2, di2, is2), (si3, di3, is3)]
    bufs = [(rows0, gs0, ss0), (rows1, gs1, ss1)]

    cid = lax.axis_index("c")
    sid = lax.axis_index("s")
    wid = cid * _NS + sid
    row0 = sid * _RPT

    if with_count:
      pltpu.sync_copy(zc_hbm, cnt)
    pltpu.sync_copy(z_hbm, acc.at[pl.ds(row0, _RPT)])

    @pl.when(sid == 0)
    def _zero_tail():
      pltpu.sync_copy(z_hbm.at[pl.ds(0, _TAIL)], acc.at[pl.ds(_TOFF, _TAIL)])

    plsc.subcore_barrier()

    def _idx(i, st):
      si, di, sem = st
      base = pl.multiple_of(wid * _EPW + i * _CH, 8)
      pltpu.async_copy(src_hbm.at[pl.ds(base, _CH)], si, sem)
      pltpu.async_copy(dst_hbm.at[pl.ds(base, _CH)], di, sem)

    def _iwait(st):
      si, di, sem = st
      pltpu.make_async_copy(src_hbm.at[pl.ds(0, _CH)], si, sem).wait()
      pltpu.make_async_copy(src_hbm.at[pl.ds(0, _CH)], di, sem).wait()

    def _gather(st, bf):
      pltpu.async_copy(p_hbm.at[st[0]], bf[0], bf[1])

    def _gwait(st, bf):
      pltpu.make_async_copy(p_hbm.at[st[0]], bf[0], bf[1]).wait()

    def _scatter(st, bf):
      pltpu.make_async_copy(bf[0], acc.at[st[1]], bf[2]).start(add=True)

    def _swait(st, bf):
      pltpu.make_async_copy(bf[0], acc.at[st[1]], bf[2]).wait()

    def _counts(st):
      if with_count:
        zero16 = jnp.zeros((16,), jnp.int32)
        one16 = jnp.ones((16,), jnp.float32)
        for g in range(_CH // 16):
          iv = st[1][pl.ds(g * 16, 16)]
          plsc.addupdate_scatter(cnt, [zero16, iv], one16)

    def slot(c, i):
      st, bf = sets[i], bufs[i % 2]
      pst, pbf = sets[(i - 1) % 4], bufs[(i - 1) % 2]

      @pl.when(c >= 2)
      def _():
        _swait(st, bf)

      @pl.when(c + 2 < _NCHUNK)
      def _():
        _idx(c + 2, sets[(i + 2) % 4])
      _iwait(st)
      _gather(st, bf)
      _counts(st)

      @pl.when(c >= 1)
      def _():
        _gwait(pst, pbf)
        _scatter(pst, pbf)

    _idx(0, sets[0])
    _idx(1, sets[1])

    def step(k, carry):
      for i in range(4):
        slot(k * 4 + i, i)
      return carry

    lax.fori_loop(0, (_NCHUNK - 1) // 4, step, 0)

    slot(_NCHUNK - 1, 0)
    _gwait(sets[0], bufs[0])
    _scatter(sets[0], bufs[0])
    _swait(sets[0], bufs[0])
    _swait(sets[3], bufs[1])
    plsc.subcore_barrier()

    pltpu.sync_copy(acc.at[pl.ds(row0, _RPT)],
                    out_hbm.at[cid, pl.ds(row0, _RPT)])
    if with_count:
      pltpu.sync_copy(cnt, cnt_hbm.at[wid])

    @pl.when(sid == 0)
    def _write_tail():
      pltpu.sync_copy(acc.at[pl.ds(_TOFF, _TAIL)],
                      out_hbm.at[cid, pl.ds(_TOFF, _TAIL)])

  kw = {}
  if with_count:
    kw["compiler_params"] = pltpu.CompilerParams(needs_layout_passes=False)
  return functools.partial(
      pl.kernel,
      out_type=out_type if with_count else out_type[0],
      mesh=mesh,
      scratch_types=scratch,
      **kw,
  )(body)


@functools.lru_cache(maxsize=None)
def _sc_agg(D, with_count):
  return _make_sc_agg(D, with_count)


def _tc1_body(x_ref, wl_ref, wr_ref, bl_ref, p_ref, r_ref):
  xv = x_ref[...]
  p_ref[...] = lax.dot_general(xv, wl_ref[...], (((1,), (1,)), ((), ())),
                               preferred_element_type=jnp.float32)
  r_ref[...] = lax.dot_general(xv, wr_ref[...], (((1,), (1,)), ((), ())),
                               preferred_element_type=jnp.float32) + bl_ref[...]


_tc1 = pl.pallas_call(
    _tc1_body,
    grid=(_GRID,),
    in_specs=[
        pl.BlockSpec((_BLK, _D_IN), lambda i: (i, 0)),
        pl.BlockSpec((_D_HID, _D_IN), lambda i: (0, 0)),
        pl.BlockSpec((_D_HID, _D_IN), lambda i: (0, 0)),
        pl.BlockSpec((1, _D_HID), lambda i: (0, 0)),
    ],
    out_specs=[
        pl.BlockSpec((_BLK, _D_HID), lambda i: (i, 0)),
        pl.BlockSpec((_BLK, _D_HID), lambda i: (i, 0)),
    ],
    out_shape=[
        jax.ShapeDtypeStruct((_N, _D_HID), jnp.float32),
        jax.ShapeDtypeStruct((_N, _D_HID), jnp.float32),
    ],
)


def _tc_cnt_body(cnt_ref, inv_ref):
  total = jnp.sum(cnt_ref[...], axis=0, keepdims=True)
  inv = 1.0 / jnp.maximum(total, 1.0)
  ones = jnp.ones((1, _D_HID), jnp.float32)
  inv_ref[...] = lax.dot_general(inv, ones, (((0,), (0,)), ((), ())),
                                 preferred_element_type=jnp.float32)


_tc_cnt = pl.pallas_call(
    _tc_cnt_body,
    in_specs=[pl.BlockSpec((_NW, _N), lambda: (0, 0))],
    out_specs=pl.BlockSpec((_N, _D_HID), lambda: (0, 0)),
    out_shape=jax.ShapeDtypeStruct((_N, _D_HID), jnp.float32),
)


def _tc2_body(agg_ref, inv_ref, r1_ref, h_ref):
  mean = (agg_ref[0] + agg_ref[1]) * inv_ref[...]
  h_ref[...] = jnp.maximum(mean + r1_ref[...], 0.0)


_tc2 = pl.pallas_call(
    _tc2_body,
    grid=(_GRID,),
    in_specs=[
        pl.BlockSpec((_NC, _BLK, _D_HID), lambda i: (0, i, 0)),
        pl.BlockSpec((_BLK, _D_HID), lambda i: (i, 0)),
        pl.BlockSpec((_BLK, _D_HID), lambda i: (i, 0)),
    ],
    out_specs=pl.BlockSpec((_BLK, _D_HID), lambda i: (i, 0)),
    out_shape=jax.ShapeDtypeStruct((_N, _D_HID), jnp.float32),
)


def _tc3_body(agg_ref, inv_ref, h_ref, wl_ref, wr_ref, bl_ref, out_ref):
  mean = (agg_ref[0] + agg_ref[1]) * inv_ref[...]
  out_ref[...] = (
      lax.dot_general(mean, wl_ref[...], (((1,), (1,)), ((), ())),
                      preferred_element_type=jnp.float32)
      + lax.dot_general(h_ref[...], wr_ref[...], (((1,), (1,)), ((), ())),
                        preferred_element_type=jnp.float32)
      + bl_ref[...])


_tc3 = pl.pallas_call(
    _tc3_body,
    grid=(_GRID,),
    in_specs=[
        pl.BlockSpec((_NC, _BLK, _D_HID), lambda i: (0, i, 0)),
        pl.BlockSpec((_BLK, _D_HID), lambda i: (i, 0)),
        pl.BlockSpec((_BLK, _D_HID), lambda i: (i, 0)),
        pl.BlockSpec((_D_OUT, _D_HID), lambda i: (0, 0)),
        pl.BlockSpec((_D_OUT, _D_HID), lambda i: (0, 0)),
        pl.BlockSpec((1, _D_OUT), lambda i: (0, 0)),
    ],
    out_specs=pl.BlockSpec((_BLK, _D_OUT), lambda i: (i, 0)),
    out_shape=jax.ShapeDtypeStruct((_N, _D_OUT), jnp.float32),
)


def kernel(x, edge_index, Wl1, bl1, Wr1, Wl2, bl2, Wr2):
  src = edge_index[0]
  dst = edge_index[1]

  z128 = jnp.zeros((_RPT, _D_HID), jnp.float32)
  zc = jnp.zeros((1, _N), jnp.float32)

  p1, r1 = _tc1(x, Wl1, Wr1, bl1.reshape(1, -1))
  agg1, cnt = _sc_agg(_D_HID, True)(p1, src, dst, z128, zc)
  invb = _tc_cnt(cnt.reshape(_NW, _N))
  h = _tc2(agg1, invb, r1)
  agg2 = _sc_agg(_D_HID, False)(h, src, dst, z128)
  return _tc3(agg2, invb, h, Wl2, Wr2, bl2.reshape(1, -1))


# speedup vs baseline: 12.3409x; 1.0192x over previous
import functools

import jax
import jax.numpy as jnp
from jax import lax
from jax.experimental import pallas as pl
from jax.experimental.pallas import tpu as pltpu
from jax.experimental.pallas import tpu_sc as plsc

_N = 10000
_E = 320000
_D_IN = 128
_D_HID = 128
_D_OUT = 64

_NC = 2
_NS = 16
_NW = _NC * _NS
_EPW = _E // _NW
_CH = 80
_NCHUNK = _EPW // _CH
_NJ = _NCHUNK // 2
_RPT = 624
_TAIL = _N - _NS * _RPT
_TOFF = _NS * _RPT

_BLK = 2000
_GRID = _N // _BLK


def _make_sc_agg(D, with_count):
  out_type = [jax.ShapeDtypeStruct((_NC, _N, D), jnp.float32)]
  scratch = (
      [pltpu.VMEM((_CH,), jnp.int32)] * 8
      + [pltpu.VMEM((_CH, D), jnp.float32)] * 2
      + [pltpu.VMEM_SHARED((_N, D), jnp.float32)]
      + [pltpu.SemaphoreType.DMA] * 8
  )
  if with_count:
    out_type.append(jax.ShapeDtypeStruct((_NW, 1, _N), jnp.float32))
    scratch.append(pltpu.VMEM((1, _N), jnp.float32))

  mesh = plsc.VectorSubcoreMesh(
      core_axis_name="c", subcore_axis_name="s",
      num_cores=_NC, num_subcores=_NS)

  def body(*refs):
    if with_count:
      (p_hbm, src_hbm, dst_hbm, z_hbm, zc_hbm,
       out_hbm, cnt_hbm,
       si0, di0, si1, di1, si2, di2, si3, di3, rows0, rows1, acc,
       is0, is1, is2, is3, gs0, gs1, ss0, ss1, cnt) = refs
    else:
      (p_hbm, src_hbm, dst_hbm, z_hbm,
       out_hbm,
       si0, di0, si1, di1, si2, di2, si3, di3, rows0, rows1, acc,
       is0, is1, is2, is3, gs0, gs1, ss0, ss1) = refs
    sets = [(si0, di0, is0), (si1, di1, is1),
            (si2, di2, is2), (si3, di3, is3)]
    bufs = [(rows0, gs0, ss0), (rows1, gs1, ss1)]

    cid = lax.axis_index("c")
    sid = lax.axis_index("s")
    wid = cid * _NS + sid
    row0 = sid * _RPT

    if with_count:
      pltpu.sync_copy(zc_hbm, cnt)
    pltpu.sync_copy(z_hbm, acc.at[pl.ds(row0, _RPT)])

    @pl.when(sid == 0)
    def _zero_tail():
      pltpu.sync_copy(z_hbm.at[pl.ds(0, _TAIL)], acc.at[pl.ds(_TOFF, _TAIL)])

    plsc.subcore_barrier()

    def _idx(i, st):
      si, di, sem = st
      base = pl.multiple_of(wid * _EPW + i * _CH, 8)
      pltpu.async_copy(src_hbm.at[pl.ds(base, _CH)], si, sem)
      pltpu.async_copy(dst_hbm.at[pl.ds(base, _CH)], di, sem)

    def _iwait(st):
      si, di, sem = st
      pltpu.make_async_copy(src_hbm.at[pl.ds(0, _CH)], si, sem).wait()
      pltpu.make_async_copy(src_hbm.at[pl.ds(0, _CH)], di, sem).wait()

    def _gather(st, bf):
      pltpu.async_copy(p_hbm.at[st[0]], bf[0], bf[1])

    def _gwait(st, bf):
      pltpu.make_async_copy(p_hbm.at[st[0]], bf[0], bf[1]).wait()

    def _scatter(st, bf):
      pltpu.make_async_copy(bf[0], acc.at[st[1]], bf[2]).start(add=True)

    def _swait(st, bf):
      pltpu.make_async_copy(bf[0], acc.at[st[1]], bf[2]).wait()

    def _counts(st):
      if with_count:
        zero16 = jnp.zeros((16,), jnp.int32)
        one16 = jnp.ones((16,), jnp.float32)
        for g in range(_CH // 16):
          iv = st[1][pl.ds(g * 16, 16)]
          plsc.addupdate_scatter(cnt, [zero16, iv], one16)

    def slot(c, i):
      st, bf = sets[i], bufs[i % 2]
      pst, pbf = sets[(i - 1) % 4], bufs[(i - 1) % 2]

      @pl.when(c >= 2)
      def _():
        _swait(st, bf)

      @pl.when(c + 2 < _NCHUNK)
      def _():
        _idx(c + 2, sets[(i + 2) % 4])
      _iwait(st)
      _gather(st, bf)
      _counts(st)

      @pl.when(c >= 1)
      def _():
        _gwait(pst, pbf)
        _scatter(pst, pbf)

    _idx(0, sets[0])
    _idx(1, sets[1])

    def step(k, carry):
      for i in range(4):
        slot(k * 4 + i, i)
      return carry

    lax.fori_loop(0, (_NCHUNK - 1) // 4, step, 0)

    slot(_NCHUNK - 1, 0)
    _gwait(sets[0], bufs[0])
    _scatter(sets[0], bufs[0])
    _swait(sets[0], bufs[0])
    _swait(sets[3], bufs[1])
    plsc.subcore_barrier()

    pltpu.sync_copy(acc.at[pl.ds(row0, _RPT)],
                    out_hbm.at[cid, pl.ds(row0, _RPT)])
    if with_count:
      pltpu.sync_copy(cnt, cnt_hbm.at[wid])

    @pl.when(sid == 0)
    def _write_tail():
      pltpu.sync_copy(acc.at[pl.ds(_TOFF, _TAIL)],
                      out_hbm.at[cid, pl.ds(_TOFF, _TAIL)])

  kw = {}
  if with_count:
    kw["compiler_params"] = pltpu.CompilerParams(needs_layout_passes=False)
  return functools.partial(
      pl.kernel,
      out_type=out_type if with_count else out_type[0],
      mesh=mesh,
      scratch_types=scratch,
      **kw,
  )(body)


@functools.lru_cache(maxsize=None)
def _sc_agg(D, with_count):
  return _make_sc_agg(D, with_count)


def _tc1_body(x_ref, wl_ref, wr_ref, bl_ref, p_ref, r_ref):
  xv = x_ref[...]
  p_ref[...] = lax.dot_general(xv, wl_ref[...], (((1,), (1,)), ((), ())),
                               preferred_element_type=jnp.float32)
  r_ref[...] = lax.dot_general(xv, wr_ref[...], (((1,), (1,)), ((), ())),
                               preferred_element_type=jnp.float32) + bl_ref[...]


_tc1 = pl.pallas_call(
    _tc1_body,
    grid=(_GRID,),
    in_specs=[
        pl.BlockSpec((_BLK, _D_IN), lambda i: (i, 0)),
        pl.BlockSpec((_D_HID, _D_IN), lambda i: (0, 0)),
        pl.BlockSpec((_D_HID, _D_IN), lambda i: (0, 0)),
        pl.BlockSpec((1, _D_HID), lambda i: (0, 0)),
    ],
    out_specs=[
        pl.BlockSpec((_BLK, _D_HID), lambda i: (i, 0)),
        pl.BlockSpec((_BLK, _D_HID), lambda i: (i, 0)),
    ],
    out_shape=[
        jax.ShapeDtypeStruct((_N, _D_HID), jnp.float32),
        jax.ShapeDtypeStruct((_N, _D_HID), jnp.float32),
    ],
)


def _inv_bcast(cnt):
  total = jnp.sum(cnt, axis=0, keepdims=True)
  inv = 1.0 / jnp.maximum(total, 1.0)
  ones = jnp.ones((1, _D_HID), jnp.float32)
  return lax.dot_general(inv, ones, (((0,), (0,)), ((), ())),
                         preferred_element_type=jnp.float32)


def _tc2_body(agg_ref, cnt_ref, r1_ref, h_ref):
  invb = _inv_bcast(cnt_ref[...])
  mean = (agg_ref[0] + agg_ref[1]) * invb
  h_ref[...] = jnp.maximum(mean + r1_ref[...], 0.0)


_tc2 = pl.pallas_call(
    _tc2_body,
    in_specs=[
        pl.BlockSpec((_NC, _N, _D_HID), lambda: (0, 0, 0)),
        pl.BlockSpec((_NW, _N), lambda: (0, 0)),
        pl.BlockSpec((_N, _D_HID), lambda: (0, 0)),
    ],
    out_specs=pl.BlockSpec((_N, _D_HID), lambda: (0, 0)),
    out_shape=jax.ShapeDtypeStruct((_N, _D_HID), jnp.float32),
)


def _tc3_body(agg_ref, cnt_ref, h_ref, wl_ref, wr_ref, bl_ref, out_ref):
  invb = _inv_bcast(cnt_ref[...])
  mean = (agg_ref[0] + agg_ref[1]) * invb
  out_ref[...] = (
      lax.dot_general(mean, wl_ref[...], (((1,), (1,)), ((), ())),
                      preferred_element_type=jnp.float32)
      + lax.dot_general(h_ref[...], wr_ref[...], (((1,), (1,)), ((), ())),
                        preferred_element_type=jnp.float32)
      + bl_ref[...])


_tc3 = pl.pallas_call(
    _tc3_body,
    in_specs=[
        pl.BlockSpec((_NC, _N, _D_HID), lambda: (0, 0, 0)),
        pl.BlockSpec((_NW, _N), lambda: (0, 0)),
        pl.BlockSpec((_N, _D_HID), lambda: (0, 0)),
        pl.BlockSpec((_D_OUT, _D_HID), lambda: (0, 0)),
        pl.BlockSpec((_D_OUT, _D_HID), lambda: (0, 0)),
        pl.BlockSpec((1, _D_OUT), lambda: (0, 0)),
    ],
    out_specs=pl.BlockSpec((_N, _D_OUT), lambda: (0, 0)),
    out_shape=jax.ShapeDtypeStruct((_N, _D_OUT), jnp.float32),
)


def kernel(x, edge_index, Wl1, bl1, Wr1, Wl2, bl2, Wr2):
  src = edge_index[0]
  dst = edge_index[1]

  z128 = jnp.zeros((_RPT, _D_HID), jnp.float32)
  zc = jnp.zeros((1, _N), jnp.float32)

  p1, r1 = _tc1(x, Wl1, Wr1, bl1.reshape(1, -1))
  agg1, cnt = _sc_agg(_D_HID, True)(p1, src, dst, z128, zc)
  cnt2 = cnt.reshape(_NW, _N)
  h = _tc2(agg1, cnt2, r1)
  agg2 = _sc_agg(_D_HID, False)(h, src, dst, z128)
  return _tc3(agg2, cnt2, h, Wl2, Wr2, bl2.reshape(1, -1))
